# Initial kernel scaffold; baseline (speedup 1.0000x reference)
import functools

import jax
import jax.numpy as jnp
from jax import lax
from jax.experimental import pallas as pl
from jax.experimental.pallas import tpu as pltpu
from jax.experimental.pallas import tpu_sc as plsc

HIDDEN = 32
LAYERS = 3
NN = 100000
NE = 1600000
D_EDGE = 4

LANES = 16
NSUB = 16
NCORE = 2
CR = 16
CE = CR * 128
R_TOTAL = 12544
NE_PAD = R_TOTAL * 128
R_TILE = R_TOTAL // NSUB
NCHUNK = R_TILE // CR
N_TILE = NN // NSUB
NPAD_ROWS = 8
DUMMY_NODE = NN

BN = 5000
NBLK = NN // BN


def _edge_body(np_hbm, snd_hbm, rcv_hbm, ef_hbm, wt_hbm, bm_hbm, z_hbm,
               agg_out, agg_sh, snd_v, rcv_v, ef_v, rows_v, wt_v, bm_v,
               gsem, ssem):
    c = lax.axis_index("c")
    s = lax.axis_index("s")

    pltpu.sync_copy(z_hbm, agg_sh.at[pl.ds(s * N_TILE, N_TILE)])

    @pl.when(s == NSUB - 1)
    def _():
        pltpu.sync_copy(z_hbm.at[pl.ds(0, NPAD_ROWS)],
                        agg_sh.at[pl.ds(NN, NPAD_ROWS)])

    pltpu.sync_copy(wt_hbm.at[c], wt_v)
    pltpu.sync_copy(bm_hbm.at[c], bm_v)
    plsc.subcore_barrier()

    w0 = wt_v[0]
    w1 = wt_v[1]
    w2 = wt_v[2]
    w3 = wt_v[3]
    bm = bm_v[...]
    base_row = s * R_TILE

    def chunk_body(i, carry):
        r0 = base_row + i * CR
        pltpu.sync_copy(snd_hbm.at[pl.ds(r0, CR)], snd_v)
        pltpu.sync_copy(rcv_hbm.at[pl.ds(r0, CR)], rcv_v)
        pltpu.sync_copy(ef_hbm.at[pl.ds(r0 * 128, CE)], ef_v)

        def addoff(j, cc):
            for k in range(8):
                sl = pl.ds(k * LANES, LANES)
                snd_v[j, sl] = snd_v[j, sl] * 2 + c
            return cc

        lax.fori_loop(0, CR, addoff, 0)

        gathers = []
        for j in range(CR):
            gathers.append(pltpu.async_copy(
                np_hbm.at[snd_v.at[j]],
                rows_v.at[pl.ds(j * 128, 128)], gsem))
        for cp in gathers:
            cp.wait()

        def edge(e, cc):
            e0 = ef_v[e, 0]
            e1 = ef_v[e, 1]
            e2 = ef_v[e, 2]
            e3 = ef_v[e, 3]
            m = rows_v[e] + (bm + e0 * w0 + e1 * w1 + e2 * w2 + e3 * w3)
            rows_v[e] = jnp.maximum(m, 0.0)
            return cc

        lax.fori_loop(0, CE, edge, 0, unroll=8)

        scatters = []
        for j in range(CR):
            scatters.append(pltpu.async_copy(
                rows_v.at[pl.ds(j * 128, 128)],
                agg_sh.at[rcv_v.at[j]], ssem, add=True))
        for cp in scatters:
            cp.wait()
        return carry

    lax.fori_loop(0, NCHUNK, chunk_body, 0)
    plsc.subcore_barrier()

    pltpu.sync_copy(agg_sh.at[pl.ds(s * N_TILE, N_TILE)],
                    agg_out.at[c, pl.ds(s * N_TILE, N_TILE)])


def _sc_edge(np_flat, snd2, rcv2, ef_pad, wt2, bm2, zrows):
    mesh = plsc.VectorSubcoreMesh(core_axis_name="c", subcore_axis_name="s")
    fn = functools.partial(
        pl.kernel,
        out_type=jax.ShapeDtypeStruct((NCORE, NN, LANES), jnp.float32),
        mesh=mesh,
        scratch_types=[
            pltpu.VMEM_SHARED((NN + NPAD_ROWS, LANES), jnp.float32),
            pltpu.VMEM((CR, 128), jnp.int32),
            pltpu.VMEM((CR, 128), jnp.int32),
            pltpu.VMEM((CE, D_EDGE), jnp.float32),
            pltpu.VMEM((CE, LANES), jnp.float32),
            pltpu.VMEM((D_EDGE, LANES), jnp.float32),
            pltpu.VMEM((LANES,), jnp.float32),
            pltpu.SemaphoreType.DMA,
            pltpu.SemaphoreType.DMA,
        ],
    )(_edge_body)
    return fn(np_flat, snd2, rcv2, ef_pad, wt2, bm2, zrows)


def _k0_body(pq, w_enc, b_enc, wm_main, wu_main, bu, ni_out, np_out, u_out):
    h = jnp.dot(pq[...], w_enc[...], preferred_element_type=jnp.float32)
    h = h + b_enc[...]
    np_out[...] = (jnp.dot(h, wm_main[...][2:], preferred_element_type=jnp.float32)
                   + wm_main[...][0:1])
    u_out[...] = (jnp.dot(h, wu_main[...][2:], preferred_element_type=jnp.float32)
                  + wu_main[...][0:1] + bu[...])
    n = h.shape[0]
    v0 = jnp.concatenate(
        [jnp.ones((n, 1), jnp.float32), jnp.zeros((n, 1), jnp.float32)], axis=1)
    ni_out[...] = jnp.concatenate([v0, h], axis=1)


def _k0(pq, w_enc, b_enc, wm_main, wu_main, bu):
    full = lambda i: (0, 0)
    return pl.pallas_call(
        _k0_body,
        grid=(NBLK,),
        in_specs=[
            pl.BlockSpec((BN, 2), lambda i: (i, 0)),
            pl.BlockSpec((2, HIDDEN), full),
            pl.BlockSpec((1, HIDDEN), full),
            pl.BlockSpec((2 + HIDDEN, HIDDEN), full),
            pl.BlockSpec((2 + HIDDEN, HIDDEN), full),
            pl.BlockSpec((1, HIDDEN), full),
        ],
        out_specs=[
            pl.BlockSpec((BN, 2 + HIDDEN), lambda i: (i, 0)),
            pl.BlockSpec((BN, HIDDEN), lambda i: (i, 0)),
            pl.BlockSpec((BN, HIDDEN), lambda i: (i, 0)),
        ],
        out_shape=[
            jax.ShapeDtypeStruct((NN, 2 + HIDDEN), jnp.float32),
            jax.ShapeDtypeStruct((NN, HIDDEN), jnp.float32),
            jax.ShapeDtypeStruct((NN, HIDDEN), jnp.float32),
        ],
    )(pq, w_enc, b_enc, wm_main, wu_main, bu)


def _kb_body(agg2, u, wu_tail, gpad, wg, bg, h1_out, g_out):
    i = pl.program_id(0)
    agg = jnp.concatenate([agg2[0], agg2[1]], axis=1)
    h1 = jnp.maximum(
        u[...] + jnp.dot(agg, wu_tail[...], preferred_element_type=jnp.float32),
        0.0)
    h1_out[...] = h1
    part = jnp.sum(h1, axis=0, keepdims=True)
    part8 = jnp.concatenate([part, jnp.zeros((7, HIDDEN), jnp.float32)], axis=0)

    @pl.when(i == 0)
    def _():
        g_out[...] = jnp.zeros_like(g_out)

    acc = g_out[...] + part8

    @pl.when(i < NBLK - 1)
    def _():
        g_out[...] = acc

    @pl.when(i == NBLK - 1)
    def _():
        mean = acc[0:1] / jnp.float32(NN)
        gin = jnp.concatenate([mean, gpad[...][0:1]], axis=1)
        gnew = jnp.maximum(
            jnp.dot(gin, wg[...], preferred_element_type=jnp.float32) + bg[...],
            0.0)
        g_out[...] = jnp.concatenate(
            [gnew, jnp.zeros((7, HIDDEN), jnp.float32)], axis=0)


def _kb(agg2, u, wu_tail, gpad, wg, bg):
    full = lambda i: (0, 0)
    return pl.pallas_call(
        _kb_body,
        grid=(NBLK,),
        in_specs=[
            pl.BlockSpec((NCORE, BN, LANES), lambda i: (0, i, 0)),
            pl.BlockSpec((BN, HIDDEN), lambda i: (i, 0)),
            pl.BlockSpec((HIDDEN, HIDDEN), full),
            pl.BlockSpec((8, HIDDEN), full),
            pl.BlockSpec((2 * HIDDEN, HIDDEN), full),
            pl.BlockSpec((1, HIDDEN), full),
        ],
        out_specs=[
            pl.BlockSpec((BN, HIDDEN), lambda i: (i, 0)),
            pl.BlockSpec((8, HIDDEN), full),
        ],
        out_shape=[
            jax.ShapeDtypeStruct((NN, HIDDEN), jnp.float32),
            jax.ShapeDtypeStruct((8, HIDDEN), jnp.float32),
        ],
    )(agg2, u, wu_tail, gpad, wg, bg)


def _kc_body_mid(h1, g2, ni, wn_main, wn_tail, bn, wdv, bdv,
                 wm_next, wu_next, bu_next, ni_out, np_out, u_out):
    gb = (jnp.dot(g2[...][0:1], wn_tail[...], preferred_element_type=jnp.float32)
          + bn[...])
    h2 = jnp.maximum(
        jnp.dot(h1[...], wn_main[...], preferred_element_type=jnp.float32) + gb,
        0.0)
    dv = jnp.dot(h2, wdv[...], preferred_element_type=jnp.float32) + bdv[...]
    vn = ni[...][:, 0:2] + dv
    nin = jnp.concatenate([vn, h2], axis=1)
    ni_out[...] = nin
    np_out[...] = jnp.dot(nin, wm_next[...], preferred_element_type=jnp.float32)
    u_out[...] = (jnp.dot(nin, wu_next[...], preferred_element_type=jnp.float32)
                  + bu_next[...])


def _kc_body_last(h1, g2, ni, wn_main, wn_tail, bn, wdv, bdv, v_out):
    gb = (jnp.dot(g2[...][0:1], wn_tail[...], preferred_element_type=jnp.float32)
          + bn[...])
    h2 = jnp.maximum(
        jnp.dot(h1[...], wn_main[...], preferred_element_type=jnp.float32) + gb,
        0.0)
    dv = jnp.dot(h2, wdv[...], preferred_element_type=jnp.float32) + bdv[...]
    v_out[...] = ni[...][:, 0:2] + dv


def _kc(h1, g2, ni, wn_main, wn_tail, bn, wdv, bdv, nxt):
    full = lambda i: (0, 0)
    in_specs = [
        pl.BlockSpec((BN, HIDDEN), lambda i: (i, 0)),
        pl.BlockSpec((8, HIDDEN), full),
        pl.BlockSpec((BN, 2 + HIDDEN), lambda i: (i, 0)),
        pl.BlockSpec((HIDDEN, HIDDEN), full),
        pl.BlockSpec((HIDDEN, HIDDEN), full),
        pl.BlockSpec((1, HIDDEN), full),
        pl.BlockSpec((HIDDEN, 2), full),
        pl.BlockSpec((1, 2), full),
    ]
    args = [h1, g2, ni, wn_main, wn_tail, bn, wdv, bdv]
    if nxt is None:
        return pl.pallas_call(
            _kc_body_last,
            grid=(NBLK,),
            in_specs=in_specs,
            out_specs=[pl.BlockSpec((BN, 2), lambda i: (i, 0))],
            out_shape=[jax.ShapeDtypeStruct((NN, 2), jnp.float32)],
        )(*args)[0]
    wm_next, wu_next, bu_next = nxt
    in_specs += [
        pl.BlockSpec((2 + HIDDEN, HIDDEN), full),
        pl.BlockSpec((2 + HIDDEN, HIDDEN), full),
        pl.BlockSpec((1, HIDDEN), full),
    ]
    args += [wm_next, wu_next, bu_next]
    return pl.pallas_call(
        _kc_body_mid,
        grid=(NBLK,),
        in_specs=in_specs,
        out_specs=[
            pl.BlockSpec((BN, 2 + HIDDEN), lambda i: (i, 0)),
            pl.BlockSpec((BN, HIDDEN), lambda i: (i, 0)),
            pl.BlockSpec((BN, HIDDEN), lambda i: (i, 0)),
        ],
        out_shape=[
            jax.ShapeDtypeStruct((NN, 2 + HIDDEN), jnp.float32),
            jax.ShapeDtypeStruct((NN, HIDDEN), jnp.float32),
            jax.ShapeDtypeStruct((NN, HIDDEN), jnp.float32),
        ],
    )(*args)


def kernel(P_Q_inj, senders, receivers, edge_features,
           W_enc, b_enc, W_msg, b_msg, W_upd, b_upd,
           W_g, b_g, W_n, b_n, W_dv, b_dv):
    f32 = jnp.float32
    node_in = 2 + HIDDEN

    pad = NE_PAD - NE
    snd_p = jnp.concatenate(
        [senders.astype(jnp.int32), jnp.zeros((pad,), jnp.int32)])
    rcv_p = jnp.concatenate(
        [receivers.astype(jnp.int32),
         jnp.full((pad,), DUMMY_NODE, jnp.int32)])
    snd2 = snd_p.reshape(R_TOTAL, 128)
    rcv2 = rcv_p.reshape(R_TOTAL, 128)
    ef_pad = jnp.concatenate(
        [edge_features.astype(f32), jnp.zeros((pad, D_EDGE), f32)], axis=0)
    zrows = jnp.zeros((N_TILE, LANES), f32)

    b2 = lambda b: b.reshape(1, -1).astype(f32)

    ni, np_l, u_l = _k0(
        P_Q_inj.astype(f32), W_enc.astype(f32), b2(b_enc),
        W_msg[0][:node_in].astype(f32), W_upd[0][:node_in].astype(f32),
        b2(b_upd[0]))

    gpad = jnp.zeros((8, HIDDEN), f32)
    v_final = None
    for l in range(LAYERS):
        wt2 = jnp.stack([W_msg[l][node_in:, :LANES],
                         W_msg[l][node_in:, LANES:]]).astype(f32)
        bm2 = b_msg[l].reshape(NCORE, LANES).astype(f32)
        np_flat = np_l.reshape(2 * NN, LANES)
        agg2 = _sc_edge(np_flat, snd2, rcv2, ef_pad, wt2, bm2, zrows)
        h1, g2 = _kb(agg2, u_l, W_upd[l][node_in:].astype(f32), gpad,
                     W_g[l].astype(f32), b2(b_g[l]))
        gpad = g2
        if l < LAYERS - 1:
            nxt = (W_msg[l + 1][:node_in].astype(f32),
                   W_upd[l + 1][:node_in].astype(f32), b2(b_upd[l + 1]))
            ni, np_l, u_l = _kc(h1, g2, ni, W_n[l][:HIDDEN].astype(f32),
                                W_n[l][HIDDEN:].astype(f32), b2(b_n[l]),
                                W_dv[l].astype(f32), b2(b_dv[l]), nxt)
        else:
            v_final = _kc(h1, g2, ni, W_n[l][:HIDDEN].astype(f32),
                          W_n[l][HIDDEN:].astype(f32), b2(b_n[l]),
                          W_dv[l].astype(f32), b2(b_dv[l]), None)
    return v_final


# speedup vs baseline: 3.2869x; 3.2869x over previous
import functools

import jax
import jax.numpy as jnp
from jax import lax
from jax.experimental import pallas as pl
from jax.experimental.pallas import tpu as pltpu
from jax.experimental.pallas import tpu_sc as plsc

HIDDEN = 32
LAYERS = 3
NN = 100000
NE = 1600000
D_EDGE = 4

LANES = 16
NSUB = 16
NCORE = 2
CR = 4
CE = CR * 128
R_TOTAL = 12544
NE_PAD = R_TOTAL * 128
R_TILE = R_TOTAL // NSUB
NCHUNK = R_TILE // CR
N_TILE = 6256
N_TILE_LAST = NN - 15 * N_TILE
NPAD_ROWS = 8
DUMMY_NODE = NN

BN = 5000
NBLK = NN // BN


def _edge_body(np_hbm, snd_hbm, rcv_hbm, ef_hbm, wt_hbm, bm_hbm, z_hbm,
               agg_out, agg_sh, snd_v, rcv_v, ef_v, rows_v, wt_v, bm_v,
               gsem, ssem):
    c = lax.axis_index("c")
    s = lax.axis_index("s")

    @pl.when(s < NSUB - 1)
    def _():
        pltpu.sync_copy(z_hbm, agg_sh.at[pl.ds(s * N_TILE, N_TILE)])

    @pl.when(s == NSUB - 1)
    def _():
        pltpu.sync_copy(z_hbm.at[pl.ds(0, N_TILE_LAST + NPAD_ROWS)],
                        agg_sh.at[pl.ds(15 * N_TILE,
                                        N_TILE_LAST + NPAD_ROWS)])

    pltpu.sync_copy(wt_hbm.at[c], wt_v)
    pltpu.sync_copy(bm_hbm.at[c], bm_v)
    plsc.subcore_barrier()

    w0 = wt_v[0]
    w1 = wt_v[1]
    w2 = wt_v[2]
    w3 = wt_v[3]
    bm = bm_v[...]
    base_row = s * R_TILE

    def chunk_body(i, carry):
        r0 = base_row + i * CR
        pltpu.sync_copy(snd_hbm.at[pl.ds(r0, CR)], snd_v)
        pltpu.sync_copy(rcv_hbm.at[pl.ds(r0, CR)], rcv_v)
        pltpu.sync_copy(ef_hbm.at[pl.ds(r0 * 32, CE // 4)], ef_v)

        def addoff(j, cc):
            for k in range(8):
                sl = pl.ds(k * LANES, LANES)
                snd_v[j, sl] = snd_v[j, sl] * 2 + c
            return cc

        lax.fori_loop(0, CR, addoff, 0)

        gathers = []
        for j in range(CR):
            gathers.append(pltpu.async_copy(
                np_hbm.at[snd_v.at[j]],
                rows_v.at[pl.ds(j * 128, 128)], gsem))
        for cp in gathers:
            cp.wait()

        def edge_group(g, cc):
            efv = ef_v[g]
            for j in range(4):
                e = g * 4 + j
                ep = (bm + efv[4 * j] * w0 + efv[4 * j + 1] * w1
                      + efv[4 * j + 2] * w2 + efv[4 * j + 3] * w3)
                rows_v[e] = jnp.maximum(rows_v[e] + ep, 0.0)
            return cc

        lax.fori_loop(0, CE // 4, edge_group, 0, unroll=4)

        scatters = []
        for j in range(CR):
            scatters.append(pltpu.async_copy(
                rows_v.at[pl.ds(j * 128, 128)],
                agg_sh.at[rcv_v.at[j]], ssem, add=True))
        for cp in scatters:
            cp.wait()
        return carry

    lax.fori_loop(0, NCHUNK, chunk_body, 0)
    plsc.subcore_barrier()

    @pl.when(s < NSUB - 1)
    def _():
        pltpu.sync_copy(agg_sh.at[pl.ds(s * N_TILE, N_TILE)],
                        agg_out.at[c, pl.ds(s * N_TILE, N_TILE)])

    @pl.when(s == NSUB - 1)
    def _():
        pltpu.sync_copy(agg_sh.at[pl.ds(15 * N_TILE, N_TILE_LAST)],
                        agg_out.at[c, pl.ds(15 * N_TILE, N_TILE_LAST)])


def _sc_edge(np_flat, snd2, rcv2, ef_pad, wt2, bm2, zrows):
    mesh = plsc.VectorSubcoreMesh(core_axis_name="c", subcore_axis_name="s")
    fn = functools.partial(
        pl.kernel,
        out_type=jax.ShapeDtypeStruct((NCORE, NN, LANES), jnp.float32),
        mesh=mesh,
        scratch_types=[
            pltpu.VMEM_SHARED((NN + NPAD_ROWS, LANES), jnp.float32),
            pltpu.VMEM((CR, 128), jnp.int32),
            pltpu.VMEM((CR, 128), jnp.int32),
            pltpu.VMEM((CE // 4, LANES), jnp.float32),
            pltpu.VMEM((CE, LANES), jnp.float32),
            pltpu.VMEM((D_EDGE, LANES), jnp.float32),
            pltpu.VMEM((LANES,), jnp.float32),
            pltpu.SemaphoreType.DMA,
            pltpu.SemaphoreType.DMA,
        ],
        compiler_params=pltpu.CompilerParams(use_tc_tiling_on_sc=False),
    )(_edge_body)
    return fn(np_flat, snd2, rcv2, ef_pad, wt2, bm2, zrows)


def _k0_body(pq, w_enc, b_enc, wm_main, wu_main, bu, ni_out, np_out, u_out):
    h = jnp.dot(pq[...], w_enc[...], preferred_element_type=jnp.float32)
    h = h + b_enc[...]
    np_out[...] = (jnp.dot(h, wm_main[...][2:], preferred_element_type=jnp.float32)
                   + wm_main[...][0:1])
    u_out[...] = (jnp.dot(h, wu_main[...][2:], preferred_element_type=jnp.float32)
                  + wu_main[...][0:1] + bu[...])
    n = h.shape[0]
    v0 = jnp.concatenate(
        [jnp.ones((n, 1), jnp.float32), jnp.zeros((n, 1), jnp.float32)], axis=1)
    ni_out[...] = jnp.concatenate([v0, h], axis=1)


def _k0(pq, w_enc, b_enc, wm_main, wu_main, bu):
    full = lambda i: (0, 0)
    return pl.pallas_call(
        _k0_body,
        grid=(NBLK,),
        in_specs=[
            pl.BlockSpec((BN, 2), lambda i: (i, 0)),
            pl.BlockSpec((2, HIDDEN), full),
            pl.BlockSpec((1, HIDDEN), full),
            pl.BlockSpec((2 + HIDDEN, HIDDEN), full),
            pl.BlockSpec((2 + HIDDEN, HIDDEN), full),
            pl.BlockSpec((1, HIDDEN), full),
        ],
        out_specs=[
            pl.BlockSpec((BN, 2 + HIDDEN), lambda i: (i, 0)),
            pl.BlockSpec((BN, HIDDEN), lambda i: (i, 0)),
            pl.BlockSpec((BN, HIDDEN), lambda i: (i, 0)),
        ],
        out_shape=[
            jax.ShapeDtypeStruct((NN, 2 + HIDDEN), jnp.float32),
            jax.ShapeDtypeStruct((NN, HIDDEN), jnp.float32),
            jax.ShapeDtypeStruct((NN, HIDDEN), jnp.float32),
        ],
    )(pq, w_enc, b_enc, wm_main, wu_main, bu)


def _kb_body(agg2, u, wu_tail, gpad, wg, bg, h1_out, g_out):
    i = pl.program_id(0)
    agg = jnp.concatenate([agg2[0], agg2[1]], axis=1)
    h1 = jnp.maximum(
        u[...] + jnp.dot(agg, wu_tail[...], preferred_element_type=jnp.float32),
        0.0)
    h1_out[...] = h1
    part = jnp.sum(h1, axis=0, keepdims=True)
    part8 = jnp.concatenate([part, jnp.zeros((7, HIDDEN), jnp.float32)], axis=0)

    @pl.when(i == 0)
    def _():
        g_out[...] = jnp.zeros_like(g_out)

    acc = g_out[...] + part8

    @pl.when(i < NBLK - 1)
    def _():
        g_out[...] = acc

    @pl.when(i == NBLK - 1)
    def _():
        mean = acc[0:1] / jnp.float32(NN)
        gin = jnp.concatenate([mean, gpad[...][0:1]], axis=1)
        gnew = jnp.maximum(
            jnp.dot(gin, wg[...], preferred_element_type=jnp.float32) + bg[...],
            0.0)
        g_out[...] = jnp.concatenate(
            [gnew, jnp.zeros((7, HIDDEN), jnp.float32)], axis=0)


def _kb(agg2, u, wu_tail, gpad, wg, bg):
    full = lambda i: (0, 0)
    return pl.pallas_call(
        _kb_body,
        grid=(NBLK,),
        in_specs=[
            pl.BlockSpec((NCORE, BN, LANES), lambda i: (0, i, 0)),
            pl.BlockSpec((BN, HIDDEN), lambda i: (i, 0)),
            pl.BlockSpec((HIDDEN, HIDDEN), full),
            pl.BlockSpec((8, HIDDEN), full),
            pl.BlockSpec((2 * HIDDEN, HIDDEN), full),
            pl.BlockSpec((1, HIDDEN), full),
        ],
        out_specs=[
            pl.BlockSpec((BN, HIDDEN), lambda i: (i, 0)),
            pl.BlockSpec((8, HIDDEN), full),
        ],
        out_shape=[
            jax.ShapeDtypeStruct((NN, HIDDEN), jnp.float32),
            jax.ShapeDtypeStruct((8, HIDDEN), jnp.float32),
        ],
    )(agg2, u, wu_tail, gpad, wg, bg)


def _kc_body_mid(h1, g2, ni, wn_main, wn_tail, bn, wdv, bdv,
                 wm_next, wu_next, bu_next, ni_out, np_out, u_out):
    gb = (jnp.dot(g2[...][0:1], wn_tail[...], preferred_element_type=jnp.float32)
          + bn[...])
    h2 = jnp.maximum(
        jnp.dot(h1[...], wn_main[...], preferred_element_type=jnp.float32) + gb,
        0.0)
    dv = jnp.dot(h2, wdv[...], preferred_element_type=jnp.float32) + bdv[...]
    vn = ni[...][:, 0:2] + dv
    nin = jnp.concatenate([vn, h2], axis=1)
    ni_out[...] = nin
    np_out[...] = jnp.dot(nin, wm_next[...], preferred_element_type=jnp.float32)
    u_out[...] = (jnp.dot(nin, wu_next[...], preferred_element_type=jnp.float32)
                  + bu_next[...])


def _kc_body_last(h1, g2, ni, wn_main, wn_tail, bn, wdv, bdv, v_out):
    gb = (jnp.dot(g2[...][0:1], wn_tail[...], preferred_element_type=jnp.float32)
          + bn[...])
    h2 = jnp.maximum(
        jnp.dot(h1[...], wn_main[...], preferred_element_type=jnp.float32) + gb,
        0.0)
    dv = jnp.dot(h2, wdv[...], preferred_element_type=jnp.float32) + bdv[...]
    v_out[...] = ni[...][:, 0:2] + dv


def _kc(h1, g2, ni, wn_main, wn_tail, bn, wdv, bdv, nxt):
    full = lambda i: (0, 0)
    in_specs = [
        pl.BlockSpec((BN, HIDDEN), lambda i: (i, 0)),
        pl.BlockSpec((8, HIDDEN), full),
        pl.BlockSpec((BN, 2 + HIDDEN), lambda i: (i, 0)),
        pl.BlockSpec((HIDDEN, HIDDEN), full),
        pl.BlockSpec((HIDDEN, HIDDEN), full),
        pl.BlockSpec((1, HIDDEN), full),
        pl.BlockSpec((HIDDEN, 2), full),
        pl.BlockSpec((1, 2), full),
    ]
    args = [h1, g2, ni, wn_main, wn_tail, bn, wdv, bdv]
    if nxt is None:
        return pl.pallas_call(
            _kc_body_last,
            grid=(NBLK,),
            in_specs=in_specs,
            out_specs=[pl.BlockSpec((BN, 2), lambda i: (i, 0))],
            out_shape=[jax.ShapeDtypeStruct((NN, 2), jnp.float32)],
        )(*args)[0]
    wm_next, wu_next, bu_next = nxt
    in_specs += [
        pl.BlockSpec((2 + HIDDEN, HIDDEN), full),
        pl.BlockSpec((2 + HIDDEN, HIDDEN), full),
        pl.BlockSpec((1, HIDDEN), full),
    ]
    args += [wm_next, wu_next, bu_next]
    return pl.pallas_call(
        _kc_body_mid,
        grid=(NBLK,),
        in_specs=in_specs,
        out_specs=[
            pl.BlockSpec((BN, 2 + HIDDEN), lambda i: (i, 0)),
            pl.BlockSpec((BN, HIDDEN), lambda i: (i, 0)),
            pl.BlockSpec((BN, HIDDEN), lambda i: (i, 0)),
        ],
        out_shape=[
            jax.ShapeDtypeStruct((NN, 2 + HIDDEN), jnp.float32),
            jax.ShapeDtypeStruct((NN, HIDDEN), jnp.float32),
            jax.ShapeDtypeStruct((NN, HIDDEN), jnp.float32),
        ],
    )(*args)


def kernel(P_Q_inj, senders, receivers, edge_features,
           W_enc, b_enc, W_msg, b_msg, W_upd, b_upd,
           W_g, b_g, W_n, b_n, W_dv, b_dv):
    f32 = jnp.float32
    node_in = 2 + HIDDEN

    pad = NE_PAD - NE
    snd_p = jnp.concatenate(
        [senders.astype(jnp.int32), jnp.zeros((pad,), jnp.int32)])
    rcv_p = jnp.concatenate(
        [receivers.astype(jnp.int32),
         jnp.full((pad,), DUMMY_NODE, jnp.int32)])
    snd2 = snd_p.reshape(R_TOTAL, 128)
    rcv2 = rcv_p.reshape(R_TOTAL, 128)
    ef_pad = jnp.concatenate(
        [edge_features.astype(f32), jnp.zeros((pad, D_EDGE), f32)],
        axis=0).reshape(NE_PAD // 4, 4 * D_EDGE)
    zrows = jnp.zeros((N_TILE, LANES), f32)

    b2 = lambda b: b.reshape(1, -1).astype(f32)

    ni, np_l, u_l = _k0(
        P_Q_inj.astype(f32), W_enc.astype(f32), b2(b_enc),
        W_msg[0][:node_in].astype(f32), W_upd[0][:node_in].astype(f32),
        b2(b_upd[0]))

    gpad = jnp.zeros((8, HIDDEN), f32)
    v_final = None
    for l in range(LAYERS):
        wt2 = jnp.stack([W_msg[l][node_in:, :LANES],
                         W_msg[l][node_in:, LANES:]]).astype(f32)
        bm2 = b_msg[l].reshape(NCORE, LANES).astype(f32)
        np_flat = np_l.reshape(2 * NN, LANES)
        agg2 = _sc_edge(np_flat, snd2, rcv2, ef_pad, wt2, bm2, zrows)
        h1, g2 = _kb(agg2, u_l, W_upd[l][node_in:].astype(f32), gpad,
                     W_g[l].astype(f32), b2(b_g[l]))
        gpad = g2
        if l < LAYERS - 1:
            nxt = (W_msg[l + 1][:node_in].astype(f32),
                   W_upd[l + 1][:node_in].astype(f32), b2(b_upd[l + 1]))
            ni, np_l, u_l = _kc(h1, g2, ni, W_n[l][:HIDDEN].astype(f32),
                                W_n[l][HIDDEN:].astype(f32), b2(b_n[l]),
                                W_dv[l].astype(f32), b2(b_dv[l]), nxt)
        else:
            v_final = _kc(h1, g2, ni, W_n[l][:HIDDEN].astype(f32),
                          W_n[l][HIDDEN:].astype(f32), b2(b_n[l]),
                          W_dv[l].astype(f32), b2(b_dv[l]), None)
    return v_final
